# baseline (speedup 1.0000x reference)
import functools

import jax
import jax.numpy as jnp
from jax import lax
from jax.experimental import pallas as pl
from jax.experimental.pallas import tpu as pltpu
from jax.experimental.pallas import tpu_sc as plsc

NC = 2
NS = 16
NW = NC * NS
LANES = 16
K = 112


def _lane_perm(vec, idx):
    dnums = lax.GatherDimensionNumbers(
        offset_dims=(), collapsed_slice_dims=(0,), start_index_map=(0,))
    return lax.gather(vec, idx, dnums, slice_sizes=(1,),
                      mode=lax.GatherScatterMode.PROMISE_IN_BOUNDS)


def _splat_lane(vec, l):
    return _lane_perm(vec, jnp.full((LANES, 1), l, jnp.int32))


def _rotate1(vec):
    idx = ((jnp.arange(LANES, dtype=jnp.int32) + 1) % LANES)[:, None]
    return _lane_perm(vec, idx)


def _sc_mesh():
    return plsc.VectorSubcoreMesh(core_axis_name="c", subcore_axis_name="s",
                                  num_cores=NC, num_subcores=NS)


def _deg_partials(dst3, ew3, n_pad, c_per_tile):
    @functools.partial(
        pl.kernel,
        out_type=jax.ShapeDtypeStruct((NC, n_pad), jnp.float32),
        mesh=_sc_mesh(),
        scratch_types=[
            pltpu.VMEM((c_per_tile, K), jnp.int32),
            pltpu.VMEM((c_per_tile, K), jnp.float32),
            pltpu.VMEM_SHARED((n_pad,), jnp.float32),
        ],
    )
    def deg_kernel(dst_hbm, ew_hbm, out_hbm, dst_v, ew_v, acc):
        cid = lax.axis_index("c")
        sid = lax.axis_index("s")
        wid = cid * NS + sid
        sl_per_tile = n_pad // NS
        base = sid * sl_per_tile

        zeros16 = jnp.zeros((LANES,), jnp.float32)

        @pl.loop(0, K // LANES)
        def _(i):
            ew_v[0, pl.ds(i * LANES, LANES)] = zeros16

        @pl.loop(0, sl_per_tile // K)
        def _(t):
            pltpu.sync_copy(ew_v.at[0], acc.at[pl.ds(base + t * K, K)])

        rem = sl_per_tile % K
        if rem:
            pltpu.sync_copy(ew_v.at[0, pl.ds(0, rem)],
                            acc.at[pl.ds(base + (sl_per_tile // K) * K, rem)])

        plsc.subcore_barrier()

        pltpu.sync_copy(dst_hbm.at[wid], dst_v)
        pltpu.sync_copy(ew_hbm.at[wid], ew_v)

        @pl.loop(0, c_per_tile)
        def _(c):
            pltpu.sync_copy(ew_v.at[c], acc.at[dst_v.at[c]], add=True)

        plsc.subcore_barrier()
        pltpu.sync_copy(acc.at[pl.ds(base, sl_per_tile)],
                        out_hbm.at[cid, pl.ds(base, sl_per_tile)])

    return deg_kernel(dst3, ew3)


RB = 3
EB = 4
UNROLL = 12


def _aggregate(hp, meta, n, d, c_per_tile):
    assert c_per_tile % UNROLL == 0
    rows_per_tile = n // NS

    @functools.partial(
        pl.kernel,
        out_type=jax.ShapeDtypeStruct((NC, NS, rows_per_tile, d), jnp.float32),
        mesh=_sc_mesh(),
        scratch_types=[
            pltpu.VMEM((1, 3, K), jnp.int32),
            pltpu.VMEM((K, d), jnp.float32),
            pltpu.VMEM_SHARED((n, d), jnp.float32),
            pltpu.SemaphoreType.DMA,
        ],
    )
    def agg_kernel(hp_hbm, meta_hbm, out_hbm,
                   meta_v, rows_v, acc, gsem):
        cid = lax.axis_index("c")
        sid = lax.axis_index("s")
        wid = cid * NS + sid
        base = sid * rows_per_tile
        zeros16 = jnp.zeros((LANES,), jnp.float32)

        @pl.loop(0, K)
        def _(e):
            for j in range(d // LANES):
                rows_v[e, pl.ds(j * LANES, LANES)] = zeros16

        @pl.loop(0, rows_per_tile // K)
        def _(t):
            pltpu.sync_copy(rows_v, acc.at[pl.ds(base + t * K, K)])

        rem = rows_per_tile % K
        if rem:
            pltpu.sync_copy(rows_v.at[pl.ds(0, rem)],
                            acc.at[pl.ds(base + (rows_per_tile // K) * K, rem)])

        plsc.subcore_barrier()

        @pl.loop(0, c_per_tile)
        def _(c):
            pltpu.sync_copy(meta_hbm.at[wid, c], meta_v.at[0])
            pltpu.async_copy(hp_hbm.at[meta_v.at[0, 0]], rows_v, gsem).wait()

            @pl.loop(0, K // LANES)
            def _(i):
                ew_vec = lax.bitcast_convert_type(
                    meta_v[0, 2, pl.ds(i * LANES, LANES)], jnp.float32)
                for l in range(LANES):
                    s = _splat_lane(ew_vec, l)
                    e = i * LANES + l
                    for j in range(d // LANES):
                        csl = pl.ds(j * LANES, LANES)
                        rows_v[e, csl] = rows_v[e, csl] * s

            pltpu.sync_copy(rows_v, acc.at[meta_v.at[0, 1]], add=True)

        plsc.subcore_barrier()
        pltpu.sync_copy(acc.at[pl.ds(base, rows_per_tile)],
                        out_hbm.at[cid, sid])

    return agg_kernel(hp, meta)


_BLK = 1000


def _dinv_block(degp_ref):
    deg = degp_ref[:, 0] + degp_ref[:, 1] + 1.0
    return lax.rsqrt(deg)[:, None]


def _prep_body(x_ref, w_ref, degp_ref, hp_ref):
    h = jnp.dot(x_ref[...], w_ref[...], preferred_element_type=jnp.float32)
    hp_ref[...] = h * _dinv_block(degp_ref)


def _mid_body(sp_ref, hp_ref, degp_ref, b_ref, w_ref, hp2_ref):
    dinv = _dinv_block(degp_ref)
    s = sp_ref[0] + sp_ref[1] + hp_ref[...]
    out1 = jnp.maximum(dinv * s + b_ref[...], 0.0)
    h2 = jnp.dot(out1, w_ref[...], preferred_element_type=jnp.float32)
    hp2_ref[...] = h2 * dinv


def _final_body(sp_ref, hp_ref, degp_ref, b_ref, out_ref):
    dinv = _dinv_block(degp_ref)
    s = sp_ref[0] + sp_ref[1] + hp_ref[...]
    out_ref[...] = jnp.maximum(dinv * s + b_ref[...], 0.0)


def _row_grid(n, d):
    grid = n // _BLK
    nd_spec = pl.BlockSpec((_BLK, d), lambda i: (i, 0))
    p_spec = pl.BlockSpec((2, _BLK, d), lambda i: (0, i, 0))
    deg_spec = pl.BlockSpec((_BLK, 2), lambda i: (i, 0))
    w_spec = pl.BlockSpec((d, d), lambda i: (0, 0))
    b_spec = pl.BlockSpec((1, d), lambda i: (0, 0))
    return grid, nd_spec, p_spec, deg_spec, w_spec, b_spec


def kernel(x, edge_idx, edge_attr, W1, b1, W2, b2):
    n, d = x.shape
    e = edge_attr.shape[0]

    chunk_all = NW * K * UNROLL
    e_pad = ((e + chunk_all - 1) // chunk_all) * chunk_all
    c_per_tile = e_pad // (NW * K)
    npg = NS * 128
    n_pad = ((n + npg - 1) // npg) * npg

    pad = e_pad - e
    src3 = jnp.pad(edge_idx[0], (0, pad)).reshape(NW, c_per_tile, K)
    dst3 = jnp.pad(edge_idx[1], (0, pad)).reshape(NW, c_per_tile, K)
    ew3 = jnp.pad(edge_attr, (0, pad)).reshape(NW, c_per_tile, K)
    meta = jnp.stack(
        [src3, dst3, lax.bitcast_convert_type(ew3, jnp.int32)], axis=2)

    degp = _deg_partials(dst3, ew3, n_pad, c_per_tile)[:, :n].T

    grid, nd_spec, p_spec, deg_spec, w_spec, b_spec = _row_grid(n, d)
    out_nd = jax.ShapeDtypeStruct((n, d), jnp.float32)

    hp1 = pl.pallas_call(
        _prep_body, grid=grid,
        in_specs=[nd_spec, w_spec, deg_spec],
        out_specs=nd_spec, out_shape=out_nd,
    )(x, W1, degp)

    s1 = _aggregate(hp1, meta, n, d, c_per_tile).reshape(NC, n, d)

    hp2 = pl.pallas_call(
        _mid_body, grid=grid,
        in_specs=[p_spec, nd_spec, deg_spec, b_spec, w_spec],
        out_specs=nd_spec, out_shape=out_nd,
    )(s1, hp1, degp, b1.reshape(1, d), W2)

    s2 = _aggregate(hp2, meta, n, d, c_per_tile).reshape(NC, n, d)

    out = pl.pallas_call(
        _final_body, grid=grid,
        in_specs=[p_spec, nd_spec, deg_spec, b_spec],
        out_specs=nd_spec, out_shape=out_nd,
    )(s2, hp2, degp, b2.reshape(1, d))

    return out


# speedup vs baseline: 3.8993x; 3.8993x over previous
import functools

import jax
import jax.numpy as jnp
from jax import lax
from jax.experimental import pallas as pl
from jax.experimental.pallas import tpu as pltpu
from jax.experimental.pallas import tpu_sc as plsc

NC = 2
NS = 16
NW = NC * NS
LANES = 16
K = 112


def _lane_perm(vec, idx):
    dnums = lax.GatherDimensionNumbers(
        offset_dims=(), collapsed_slice_dims=(0,), start_index_map=(0,))
    return lax.gather(vec, idx, dnums, slice_sizes=(1,),
                      mode=lax.GatherScatterMode.PROMISE_IN_BOUNDS)


def _splat_lane(vec, l):
    return _lane_perm(vec, jnp.full((LANES, 1), l, jnp.int32))


def _rotate1(vec):
    idx = ((jnp.arange(LANES, dtype=jnp.int32) + 1) % LANES)[:, None]
    return _lane_perm(vec, idx)


def _sc_mesh():
    return plsc.VectorSubcoreMesh(core_axis_name="c", subcore_axis_name="s",
                                  num_cores=NC, num_subcores=NS)


def _deg_partials(dst3, ew3, n_pad, c_per_tile):
    @functools.partial(
        pl.kernel,
        out_type=jax.ShapeDtypeStruct((NC, n_pad), jnp.float32),
        mesh=_sc_mesh(),
        scratch_types=[
            pltpu.VMEM((c_per_tile, K), jnp.int32),
            pltpu.VMEM((c_per_tile, K), jnp.float32),
            pltpu.VMEM_SHARED((n_pad,), jnp.float32),
        ],
    )
    def deg_kernel(dst_hbm, ew_hbm, out_hbm, dst_v, ew_v, acc):
        cid = lax.axis_index("c")
        sid = lax.axis_index("s")
        wid = cid * NS + sid
        sl_per_tile = n_pad // NS
        base = sid * sl_per_tile

        zeros16 = jnp.zeros((LANES,), jnp.float32)

        @pl.loop(0, K // LANES)
        def _(i):
            ew_v[0, pl.ds(i * LANES, LANES)] = zeros16

        @pl.loop(0, sl_per_tile // K)
        def _(t):
            pltpu.sync_copy(ew_v.at[0], acc.at[pl.ds(base + t * K, K)])

        rem = sl_per_tile % K
        if rem:
            pltpu.sync_copy(ew_v.at[0, pl.ds(0, rem)],
                            acc.at[pl.ds(base + (sl_per_tile // K) * K, rem)])

        plsc.subcore_barrier()

        pltpu.sync_copy(dst_hbm.at[wid], dst_v)
        pltpu.sync_copy(ew_hbm.at[wid], ew_v)

        @pl.loop(0, c_per_tile)
        def _(c):
            pltpu.sync_copy(ew_v.at[c], acc.at[dst_v.at[c]], add=True)

        plsc.subcore_barrier()
        pltpu.sync_copy(acc.at[pl.ds(base, sl_per_tile)],
                        out_hbm.at[cid, pl.ds(base, sl_per_tile)])

    return deg_kernel(dst3, ew3)


RB = 3
EB = 4
UNROLL = 1


def _aggregate(hp, meta, n, d, c_per_tile):
    assert c_per_tile % UNROLL == 0
    rows_per_tile = n // NS

    @functools.partial(
        pl.kernel,
        out_type=jax.ShapeDtypeStruct((NC, NS, rows_per_tile, d), jnp.float32),
        mesh=_sc_mesh(),
        scratch_types=[
            pltpu.VMEM((1, 3, K), jnp.int32),
            pltpu.VMEM((K, d), jnp.float32),
            pltpu.VMEM_SHARED((n, d), jnp.float32),
            pltpu.SemaphoreType.DMA,
        ],
    )
    def agg_kernel(hp_hbm, meta_hbm, out_hbm,
                   meta_v, rows_v, acc, gsem):
        cid = lax.axis_index("c")
        sid = lax.axis_index("s")
        wid = cid * NS + sid
        base = sid * rows_per_tile
        zeros16 = jnp.zeros((LANES,), jnp.float32)

        @pl.loop(0, K)
        def _(e):
            for j in range(d // LANES):
                rows_v[e, pl.ds(j * LANES, LANES)] = zeros16

        @pl.loop(0, rows_per_tile // K)
        def _(t):
            pltpu.sync_copy(rows_v, acc.at[pl.ds(base + t * K, K)])

        rem = rows_per_tile % K
        if rem:
            pltpu.sync_copy(rows_v.at[pl.ds(0, rem)],
                            acc.at[pl.ds(base + (rows_per_tile // K) * K, rem)])

        plsc.subcore_barrier()

        @pl.loop(0, c_per_tile)
        def _(c):
            pltpu.sync_copy(meta_hbm.at[wid, c], meta_v.at[0])
            pltpu.async_copy(hp_hbm.at[meta_v.at[0, 0]], rows_v, gsem).wait()

            @pl.loop(0, K // LANES)
            def _(i):
                ew_vec = lax.bitcast_convert_type(
                    meta_v[0, 2, pl.ds(i * LANES, LANES)], jnp.float32)
                for l in range(LANES):
                    s = _splat_lane(ew_vec, l)
                    e = i * LANES + l
                    for j in range(d // LANES):
                        csl = pl.ds(j * LANES, LANES)
                        rows_v[e, csl] = rows_v[e, csl] * s

            pltpu.sync_copy(rows_v, acc.at[meta_v.at[0, 1]], add=True)

        plsc.subcore_barrier()
        pltpu.sync_copy(acc.at[pl.ds(base, rows_per_tile)],
                        out_hbm.at[cid, sid])

    return agg_kernel(hp, meta)


_BLK = 1000


def _dinv_block(degp_ref):
    deg = degp_ref[:, 0] + degp_ref[:, 1] + 1.0
    return lax.rsqrt(deg)[:, None]


def _prep_body(x_ref, w_ref, degp_ref, hp_ref):
    h = jnp.dot(x_ref[...], w_ref[...], preferred_element_type=jnp.float32)
    hp_ref[...] = h * _dinv_block(degp_ref)


def _mid_body(sp_ref, hp_ref, degp_ref, b_ref, w_ref, hp2_ref):
    dinv = _dinv_block(degp_ref)
    s = sp_ref[0] + sp_ref[1] + hp_ref[...]
    out1 = jnp.maximum(dinv * s + b_ref[...], 0.0)
    h2 = jnp.dot(out1, w_ref[...], preferred_element_type=jnp.float32)
    hp2_ref[...] = h2 * dinv


def _final_body(sp_ref, hp_ref, degp_ref, b_ref, out_ref):
    dinv = _dinv_block(degp_ref)
    s = sp_ref[0] + sp_ref[1] + hp_ref[...]
    out_ref[...] = jnp.maximum(dinv * s + b_ref[...], 0.0)


def _row_grid(n, d):
    grid = n // _BLK
    nd_spec = pl.BlockSpec((_BLK, d), lambda i: (i, 0))
    p_spec = pl.BlockSpec((2, _BLK, d), lambda i: (0, i, 0))
    deg_spec = pl.BlockSpec((_BLK, 2), lambda i: (i, 0))
    w_spec = pl.BlockSpec((d, d), lambda i: (0, 0))
    b_spec = pl.BlockSpec((1, d), lambda i: (0, 0))
    return grid, nd_spec, p_spec, deg_spec, w_spec, b_spec


def kernel(x, edge_idx, edge_attr, W1, b1, W2, b2):
    n, d = x.shape
    e = edge_attr.shape[0]

    chunk_all = NW * K * UNROLL
    e_pad = ((e + chunk_all - 1) // chunk_all) * chunk_all
    c_per_tile = e_pad // (NW * K)
    npg = NS * 128
    n_pad = ((n + npg - 1) // npg) * npg

    pad = e_pad - e
    pad_idx = jnp.arange(pad, dtype=jnp.int32) % n
    src3 = jnp.concatenate([edge_idx[0], pad_idx]).reshape(NW, c_per_tile, K)
    dst3 = jnp.concatenate([edge_idx[1], pad_idx]).reshape(NW, c_per_tile, K)
    ew3 = jnp.concatenate(
        [edge_attr, jnp.zeros((pad,), jnp.float32)]).reshape(NW, c_per_tile, K)
    meta = jnp.stack(
        [src3, dst3, lax.bitcast_convert_type(ew3, jnp.int32)], axis=2)

    degp = _deg_partials(dst3, ew3, n_pad, c_per_tile)[:, :n].T

    grid, nd_spec, p_spec, deg_spec, w_spec, b_spec = _row_grid(n, d)
    out_nd = jax.ShapeDtypeStruct((n, d), jnp.float32)

    hp1 = pl.pallas_call(
        _prep_body, grid=grid,
        in_specs=[nd_spec, w_spec, deg_spec],
        out_specs=nd_spec, out_shape=out_nd,
    )(x, W1, degp)

    s1 = _aggregate(hp1, meta, n, d, c_per_tile).reshape(NC, n, d)

    hp2 = pl.pallas_call(
        _mid_body, grid=grid,
        in_specs=[p_spec, nd_spec, deg_spec, b_spec, w_spec],
        out_specs=nd_spec, out_shape=out_nd,
    )(s1, hp1, degp, b1.reshape(1, d), W2)

    s2 = _aggregate(hp2, meta, n, d, c_per_tile).reshape(NC, n, d)

    out = pl.pallas_call(
        _final_body, grid=grid,
        in_specs=[p_spec, nd_spec, deg_spec, b_spec],
        out_specs=nd_spec, out_shape=out_nd,
    )(s2, hp2, degp, b2.reshape(1, d))

    return out


# speedup vs baseline: 6.9818x; 1.7905x over previous
import functools

import jax
import jax.numpy as jnp
from jax import lax
from jax.experimental import pallas as pl
from jax.experimental.pallas import tpu as pltpu
from jax.experimental.pallas import tpu_sc as plsc

NC = 2
NS = 16
NW = NC * NS
LANES = 16
K = 112


def _lane_perm(vec, idx):
    dnums = lax.GatherDimensionNumbers(
        offset_dims=(), collapsed_slice_dims=(0,), start_index_map=(0,))
    return lax.gather(vec, idx, dnums, slice_sizes=(1,),
                      mode=lax.GatherScatterMode.PROMISE_IN_BOUNDS)


def _splat_lane(vec, l):
    return _lane_perm(vec, jnp.full((LANES, 1), l, jnp.int32))


def _rotate1(vec):
    idx = ((jnp.arange(LANES, dtype=jnp.int32) + 1) % LANES)[:, None]
    return _lane_perm(vec, idx)


def _sc_mesh():
    return plsc.VectorSubcoreMesh(core_axis_name="c", subcore_axis_name="s",
                                  num_cores=NC, num_subcores=NS)


def _deg_partials(dst3, ew3, n_pad, c_per_tile):
    @functools.partial(
        pl.kernel,
        out_type=jax.ShapeDtypeStruct((NC, n_pad), jnp.float32),
        mesh=_sc_mesh(),
        scratch_types=[
            pltpu.VMEM((c_per_tile, K), jnp.int32),
            pltpu.VMEM((c_per_tile, K), jnp.float32),
            pltpu.VMEM_SHARED((n_pad,), jnp.float32),
        ],
    )
    def deg_kernel(dst_hbm, ew_hbm, out_hbm, dst_v, ew_v, acc):
        cid = lax.axis_index("c")
        sid = lax.axis_index("s")
        wid = cid * NS + sid
        sl_per_tile = n_pad // NS
        base = sid * sl_per_tile

        zeros16 = jnp.zeros((LANES,), jnp.float32)

        @pl.loop(0, K // LANES)
        def _(i):
            ew_v[0, pl.ds(i * LANES, LANES)] = zeros16

        @pl.loop(0, sl_per_tile // K)
        def _(t):
            pltpu.sync_copy(ew_v.at[0], acc.at[pl.ds(base + t * K, K)])

        rem = sl_per_tile % K
        if rem:
            pltpu.sync_copy(ew_v.at[0, pl.ds(0, rem)],
                            acc.at[pl.ds(base + (sl_per_tile // K) * K, rem)])

        plsc.subcore_barrier()

        pltpu.sync_copy(dst_hbm.at[wid], dst_v)
        pltpu.sync_copy(ew_hbm.at[wid], ew_v)

        @pl.loop(0, c_per_tile)
        def _(c):
            pltpu.sync_copy(ew_v.at[c], acc.at[dst_v.at[c]], add=True)

        plsc.subcore_barrier()
        pltpu.sync_copy(acc.at[pl.ds(base, sl_per_tile)],
                        out_hbm.at[cid, pl.ds(base, sl_per_tile)])

    return deg_kernel(dst3, ew3)


RB = 3
EB = 4
UNROLL = 12


def _aggregate(hp, meta, n, d, c_per_tile):
    assert c_per_tile % UNROLL == 0
    rows_per_tile = n // NS

    @functools.partial(
        pl.kernel,
        out_type=jax.ShapeDtypeStruct((NC, NS, rows_per_tile, d), jnp.float32),
        mesh=_sc_mesh(),
        scratch_types=[
            pltpu.VMEM((EB, 3, K), jnp.int32),
            [pltpu.VMEM((K, d), jnp.float32)] * RB,
            pltpu.VMEM_SHARED((n, d), jnp.float32),
            [pltpu.SemaphoreType.DMA] * EB,
            [pltpu.SemaphoreType.DMA] * RB,
            [pltpu.SemaphoreType.DMA] * RB,
        ],
    )
    def agg_kernel(hp_hbm, meta_hbm, out_hbm,
                   meta_v, rows, acc, esem, gsem, ssem):
        rows_v = rows[0]
        cid = lax.axis_index("c")
        sid = lax.axis_index("s")
        wid = cid * NS + sid
        base = sid * rows_per_tile
        zeros16 = jnp.zeros((LANES,), jnp.float32)

        @pl.loop(0, K)
        def _(e):
            for j in range(d // LANES):
                rows_v[e, pl.ds(j * LANES, LANES)] = zeros16

        @pl.loop(0, rows_per_tile // K)
        def _(t):
            pltpu.sync_copy(rows_v, acc.at[pl.ds(base + t * K, K)])

        rem = rows_per_tile % K
        if rem:
            pltpu.sync_copy(rows_v.at[pl.ds(0, rem)],
                            acc.at[pl.ds(base + (rows_per_tile // K) * K, rem)])

        plsc.subcore_barrier()

        def edges_desc(c, sl):
            return pltpu.make_async_copy(meta_hbm.at[wid, c], meta_v.at[sl],
                                         esem[sl])

        def gather_desc(b, sl):
            return pltpu.make_async_copy(hp_hbm.at[meta_v.at[sl, 0]], rows[b],
                                         gsem[b])

        def scatter_desc(b, sl):
            return pltpu.make_async_copy(rows[b], acc.at[meta_v.at[sl, 1]],
                                         ssem[b])

        edges_desc(0, 0).start()
        edges_desc(1, 1).start()
        edges_desc(0, 0).wait()
        gather_desc(0, 0).start()

        @pl.loop(0, c_per_tile // UNROLL)
        def _(t):
            for r in range(UNROLL):
                b = r % RB
                bn = (r + 1) % RB
                sl = r % EB
                sln = (r + 1) % EB
                sl2 = (r + 2) % EB
                c = t * UNROLL + r

                @pl.when(c >= 2)
                def _():
                    scatter_desc(bn, sl2).wait()

                @pl.when(c + 2 < c_per_tile)
                def _():
                    edges_desc(c + 2, sl2).start()

                @pl.when(c + 1 < c_per_tile)
                def _():
                    edges_desc(c + 1, sln).wait()
                    gather_desc(bn, sln).start()

                gather_desc(b, sl).wait()

                @pl.loop(0, K // LANES)
                def _(i):
                    ew_vec = lax.bitcast_convert_type(
                        meta_v[sl, 2, pl.ds(i * LANES, LANES)], jnp.float32)
                    for l in range(LANES):
                        s = _splat_lane(ew_vec, l)
                        e = i * LANES + l
                        for j in range(d // LANES):
                            csl = pl.ds(j * LANES, LANES)
                            rows[b][e, csl] = rows[b][e, csl] * s

                scatter_desc(b, sl).start(add=True)

        for c_last in (c_per_tile - 2, c_per_tile - 1):
            scatter_desc(c_last % RB, c_last % EB).wait()

        plsc.subcore_barrier()
        pltpu.sync_copy(acc.at[pl.ds(base, rows_per_tile)],
                        out_hbm.at[cid, sid])

    return agg_kernel(hp, meta)


_BLK = 1000


def _dinv_block(degp_ref):
    deg = degp_ref[:, 0] + degp_ref[:, 1] + 1.0
    return lax.rsqrt(deg)[:, None]


def _prep_body(x_ref, w_ref, degp_ref, hp_ref):
    h = jnp.dot(x_ref[...], w_ref[...], preferred_element_type=jnp.float32)
    hp_ref[...] = h * _dinv_block(degp_ref)


def _mid_body(sp_ref, hp_ref, degp_ref, b_ref, w_ref, hp2_ref):
    dinv = _dinv_block(degp_ref)
    s = sp_ref[0] + sp_ref[1] + hp_ref[...]
    out1 = jnp.maximum(dinv * s + b_ref[...], 0.0)
    h2 = jnp.dot(out1, w_ref[...], preferred_element_type=jnp.float32)
    hp2_ref[...] = h2 * dinv


def _final_body(sp_ref, hp_ref, degp_ref, b_ref, out_ref):
    dinv = _dinv_block(degp_ref)
    s = sp_ref[0] + sp_ref[1] + hp_ref[...]
    out_ref[...] = jnp.maximum(dinv * s + b_ref[...], 0.0)


def _row_grid(n, d):
    grid = n // _BLK
    nd_spec = pl.BlockSpec((_BLK, d), lambda i: (i, 0))
    p_spec = pl.BlockSpec((2, _BLK, d), lambda i: (0, i, 0))
    deg_spec = pl.BlockSpec((_BLK, 2), lambda i: (i, 0))
    w_spec = pl.BlockSpec((d, d), lambda i: (0, 0))
    b_spec = pl.BlockSpec((1, d), lambda i: (0, 0))
    return grid, nd_spec, p_spec, deg_spec, w_spec, b_spec


def kernel(x, edge_idx, edge_attr, W1, b1, W2, b2):
    n, d = x.shape
    e = edge_attr.shape[0]

    chunk_all = NW * K * UNROLL
    e_pad = ((e + chunk_all - 1) // chunk_all) * chunk_all
    c_per_tile = e_pad // (NW * K)
    npg = NS * 128
    n_pad = ((n + npg - 1) // npg) * npg

    pad = e_pad - e
    pad_idx = jnp.arange(pad, dtype=jnp.int32) % n
    src3 = jnp.concatenate([edge_idx[0], pad_idx]).reshape(NW, c_per_tile, K)
    dst3 = jnp.concatenate([edge_idx[1], pad_idx]).reshape(NW, c_per_tile, K)
    ew3 = jnp.concatenate(
        [edge_attr, jnp.zeros((pad,), jnp.float32)]).reshape(NW, c_per_tile, K)
    meta = jnp.stack(
        [src3, dst3, lax.bitcast_convert_type(ew3, jnp.int32)], axis=2)

    degp = _deg_partials(dst3, ew3, n_pad, c_per_tile)[:, :n].T

    grid, nd_spec, p_spec, deg_spec, w_spec, b_spec = _row_grid(n, d)
    out_nd = jax.ShapeDtypeStruct((n, d), jnp.float32)

    hp1 = pl.pallas_call(
        _prep_body, grid=grid,
        in_specs=[nd_spec, w_spec, deg_spec],
        out_specs=nd_spec, out_shape=out_nd,
    )(x, W1, degp)

    s1 = _aggregate(hp1, meta, n, d, c_per_tile).reshape(NC, n, d)

    hp2 = pl.pallas_call(
        _mid_body, grid=grid,
        in_specs=[p_spec, nd_spec, deg_spec, b_spec, w_spec],
        out_specs=nd_spec, out_shape=out_nd,
    )(s1, hp1, degp, b1.reshape(1, d), W2)

    s2 = _aggregate(hp2, meta, n, d, c_per_tile).reshape(NC, n, d)

    out = pl.pallas_call(
        _final_body, grid=grid,
        in_specs=[p_spec, nd_spec, deg_spec, b_spec],
        out_specs=nd_spec, out_shape=out_nd,
    )(s2, hp2, degp, b2.reshape(1, d))

    return out


# speedup vs baseline: 7.0865x; 1.0150x over previous
import functools

import jax
import jax.numpy as jnp
from jax import lax
from jax.experimental import pallas as pl
from jax.experimental.pallas import tpu as pltpu
from jax.experimental.pallas import tpu_sc as plsc

NC = 2
NS = 16
NW = NC * NS
LANES = 16
K = 80


def _lane_perm(vec, idx):
    dnums = lax.GatherDimensionNumbers(
        offset_dims=(), collapsed_slice_dims=(0,), start_index_map=(0,))
    return lax.gather(vec, idx, dnums, slice_sizes=(1,),
                      mode=lax.GatherScatterMode.PROMISE_IN_BOUNDS)


def _splat_lane(vec, l):
    return _lane_perm(vec, jnp.full((LANES, 1), l, jnp.int32))


def _rotate1(vec):
    idx = ((jnp.arange(LANES, dtype=jnp.int32) + 1) % LANES)[:, None]
    return _lane_perm(vec, idx)


def _sc_mesh():
    return plsc.VectorSubcoreMesh(core_axis_name="c", subcore_axis_name="s",
                                  num_cores=NC, num_subcores=NS)


def _deg_partials(dst3, ew3, n_pad, c_per_tile):
    @functools.partial(
        pl.kernel,
        out_type=jax.ShapeDtypeStruct((NC, n_pad), jnp.float32),
        mesh=_sc_mesh(),
        scratch_types=[
            pltpu.VMEM((c_per_tile, K), jnp.int32),
            pltpu.VMEM((c_per_tile, K), jnp.float32),
            pltpu.VMEM_SHARED((n_pad,), jnp.float32),
        ],
    )
    def deg_kernel(dst_hbm, ew_hbm, out_hbm, dst_v, ew_v, acc):
        cid = lax.axis_index("c")
        sid = lax.axis_index("s")
        wid = cid * NS + sid
        sl_per_tile = n_pad // NS
        base = sid * sl_per_tile

        zeros16 = jnp.zeros((LANES,), jnp.float32)

        @pl.loop(0, K // LANES)
        def _(i):
            ew_v[0, pl.ds(i * LANES, LANES)] = zeros16

        @pl.loop(0, sl_per_tile // K)
        def _(t):
            pltpu.sync_copy(ew_v.at[0], acc.at[pl.ds(base + t * K, K)])

        rem = sl_per_tile % K
        if rem:
            pltpu.sync_copy(ew_v.at[0, pl.ds(0, rem)],
                            acc.at[pl.ds(base + (sl_per_tile // K) * K, rem)])

        plsc.subcore_barrier()

        pltpu.sync_copy(dst_hbm.at[wid], dst_v)
        pltpu.sync_copy(ew_hbm.at[wid], ew_v)

        @pl.loop(0, c_per_tile)
        def _(c):
            pltpu.sync_copy(ew_v.at[c], acc.at[dst_v.at[c]], add=True)

        plsc.subcore_barrier()
        pltpu.sync_copy(acc.at[pl.ds(base, sl_per_tile)],
                        out_hbm.at[cid, pl.ds(base, sl_per_tile)])

    return deg_kernel(dst3, ew3)


RB = 4
EB = 6
UNROLL = 12


def _aggregate(hp, meta, n, d, c_per_tile):
    assert c_per_tile % UNROLL == 0
    rows_per_tile = n // NS

    @functools.partial(
        pl.kernel,
        out_type=jax.ShapeDtypeStruct((NC, NS, rows_per_tile, d), jnp.float32),
        mesh=_sc_mesh(),
        scratch_types=[
            pltpu.VMEM((EB, 3, K), jnp.int32),
            [pltpu.VMEM((K, d), jnp.float32)] * RB,
            pltpu.VMEM_SHARED((n, d), jnp.float32),
            [pltpu.SemaphoreType.DMA] * EB,
            [pltpu.SemaphoreType.DMA] * RB,
            [pltpu.SemaphoreType.DMA] * RB,
        ],
    )
    def agg_kernel(hp_hbm, meta_hbm, out_hbm,
                   meta_v, rows, acc, esem, gsem, ssem):
        rows_v = rows[0]
        cid = lax.axis_index("c")
        sid = lax.axis_index("s")
        wid = cid * NS + sid
        base = sid * rows_per_tile
        zeros16 = jnp.zeros((LANES,), jnp.float32)

        @pl.loop(0, K)
        def _(e):
            for j in range(d // LANES):
                rows_v[e, pl.ds(j * LANES, LANES)] = zeros16

        @pl.loop(0, rows_per_tile // K)
        def _(t):
            pltpu.sync_copy(rows_v, acc.at[pl.ds(base + t * K, K)])

        rem = rows_per_tile % K
        if rem:
            pltpu.sync_copy(rows_v.at[pl.ds(0, rem)],
                            acc.at[pl.ds(base + (rows_per_tile // K) * K, rem)])

        plsc.subcore_barrier()

        def edges_desc(c, sl):
            return pltpu.make_async_copy(meta_hbm.at[wid, c], meta_v.at[sl],
                                         esem[sl])

        def gather_desc(b, sl):
            return pltpu.make_async_copy(hp_hbm.at[meta_v.at[sl, 0]], rows[b],
                                         gsem[b])

        def scatter_desc(b, sl):
            return pltpu.make_async_copy(rows[b], acc.at[meta_v.at[sl, 1]],
                                         ssem[b])

        edges_desc(0, 0).start()
        edges_desc(1, 1).start()
        edges_desc(2, 2).start()
        edges_desc(0, 0).wait()
        gather_desc(0, 0).start()
        edges_desc(1, 1).wait()
        gather_desc(1, 1).start()

        @pl.loop(0, c_per_tile // UNROLL)
        def _(t):
            for r in range(UNROLL):
                b = r % RB
                b2 = (r + 2) % RB
                sl = r % EB
                sl2 = (r + 2) % EB
                sl3 = (r + 3) % EB
                c = t * UNROLL + r

                @pl.when(c >= 2)
                def _():
                    scatter_desc(b2, sl2).wait()

                @pl.when(c + 2 < c_per_tile)
                def _():
                    edges_desc(c + 2, sl2).wait()
                    gather_desc(b2, sl2).start()

                @pl.when(c + 3 < c_per_tile)
                def _():
                    edges_desc(c + 3, sl3).start()

                gather_desc(b, sl).wait()

                @pl.loop(0, K // LANES)
                def _(i):
                    ew_vec = lax.bitcast_convert_type(
                        meta_v[sl, 2, pl.ds(i * LANES, LANES)], jnp.float32)
                    for l in range(LANES):
                        s = _splat_lane(ew_vec, l)
                        e = i * LANES + l
                        for j in range(d // LANES):
                            csl = pl.ds(j * LANES, LANES)
                            rows[b][e, csl] = rows[b][e, csl] * s

                scatter_desc(b, sl).start(add=True)

        for c_last in (c_per_tile - 2, c_per_tile - 1):
            scatter_desc(c_last % RB, c_last % EB).wait()

        plsc.subcore_barrier()
        pltpu.sync_copy(acc.at[pl.ds(base, rows_per_tile)],
                        out_hbm.at[cid, sid])

    return agg_kernel(hp, meta)


_BLK = 1000


def _dinv_block(degp_ref):
    deg = degp_ref[:, 0] + degp_ref[:, 1] + 1.0
    return lax.rsqrt(deg)[:, None]


def _prep_body(x_ref, w_ref, degp_ref, hp_ref):
    h = jnp.dot(x_ref[...], w_ref[...], preferred_element_type=jnp.float32)
    hp_ref[...] = h * _dinv_block(degp_ref)


def _mid_body(sp_ref, hp_ref, degp_ref, b_ref, w_ref, hp2_ref):
    dinv = _dinv_block(degp_ref)
    s = sp_ref[0] + sp_ref[1] + hp_ref[...]
    out1 = jnp.maximum(dinv * s + b_ref[...], 0.0)
    h2 = jnp.dot(out1, w_ref[...], preferred_element_type=jnp.float32)
    hp2_ref[...] = h2 * dinv


def _final_body(sp_ref, hp_ref, degp_ref, b_ref, out_ref):
    dinv = _dinv_block(degp_ref)
    s = sp_ref[0] + sp_ref[1] + hp_ref[...]
    out_ref[...] = jnp.maximum(dinv * s + b_ref[...], 0.0)


def _row_grid(n, d):
    grid = n // _BLK
    nd_spec = pl.BlockSpec((_BLK, d), lambda i: (i, 0))
    p_spec = pl.BlockSpec((2, _BLK, d), lambda i: (0, i, 0))
    deg_spec = pl.BlockSpec((_BLK, 2), lambda i: (i, 0))
    w_spec = pl.BlockSpec((d, d), lambda i: (0, 0))
    b_spec = pl.BlockSpec((1, d), lambda i: (0, 0))
    return grid, nd_spec, p_spec, deg_spec, w_spec, b_spec


def kernel(x, edge_idx, edge_attr, W1, b1, W2, b2):
    n, d = x.shape
    e = edge_attr.shape[0]

    chunk_all = NW * K * UNROLL
    e_pad = ((e + chunk_all - 1) // chunk_all) * chunk_all
    c_per_tile = e_pad // (NW * K)
    npg = NS * 128
    n_pad = ((n + npg - 1) // npg) * npg

    pad = e_pad - e
    pad_idx = jnp.arange(pad, dtype=jnp.int32) % n
    src3 = jnp.concatenate([edge_idx[0], pad_idx]).reshape(NW, c_per_tile, K)
    dst3 = jnp.concatenate([edge_idx[1], pad_idx]).reshape(NW, c_per_tile, K)
    ew3 = jnp.concatenate(
        [edge_attr, jnp.zeros((pad,), jnp.float32)]).reshape(NW, c_per_tile, K)
    meta = jnp.stack(
        [src3, dst3, lax.bitcast_convert_type(ew3, jnp.int32)], axis=2)

    degp = _deg_partials(dst3, ew3, n_pad, c_per_tile)[:, :n].T

    grid, nd_spec, p_spec, deg_spec, w_spec, b_spec = _row_grid(n, d)
    out_nd = jax.ShapeDtypeStruct((n, d), jnp.float32)

    hp1 = pl.pallas_call(
        _prep_body, grid=grid,
        in_specs=[nd_spec, w_spec, deg_spec],
        out_specs=nd_spec, out_shape=out_nd,
    )(x, W1, degp)

    s1 = _aggregate(hp1, meta, n, d, c_per_tile).reshape(NC, n, d)

    hp2 = pl.pallas_call(
        _mid_body, grid=grid,
        in_specs=[p_spec, nd_spec, deg_spec, b_spec, w_spec],
        out_specs=nd_spec, out_shape=out_nd,
    )(s1, hp1, degp, b1.reshape(1, d), W2)

    s2 = _aggregate(hp2, meta, n, d, c_per_tile).reshape(NC, n, d)

    out = pl.pallas_call(
        _final_body, grid=grid,
        in_specs=[p_spec, nd_spec, deg_spec, b_spec],
        out_specs=nd_spec, out_shape=out_nd,
    )(s2, hp2, degp, b2.reshape(1, d))

    return out


# speedup vs baseline: 7.1865x; 1.0141x over previous
import functools

import jax
import jax.numpy as jnp
from jax import lax
from jax.experimental import pallas as pl
from jax.experimental.pallas import tpu as pltpu
from jax.experimental.pallas import tpu_sc as plsc

NC = 2
NS = 16
NW = NC * NS
LANES = 16
K = 80


def _lane_perm(vec, idx):
    dnums = lax.GatherDimensionNumbers(
        offset_dims=(), collapsed_slice_dims=(0,), start_index_map=(0,))
    return lax.gather(vec, idx, dnums, slice_sizes=(1,),
                      mode=lax.GatherScatterMode.PROMISE_IN_BOUNDS)


def _splat_lane(vec, l):
    return _lane_perm(vec, jnp.full((LANES, 1), l, jnp.int32))


def _rotate1(vec):
    idx = ((jnp.arange(LANES, dtype=jnp.int32) + 1) % LANES)[:, None]
    return _lane_perm(vec, idx)


def _sc_mesh():
    return plsc.VectorSubcoreMesh(core_axis_name="c", subcore_axis_name="s",
                                  num_cores=NC, num_subcores=NS)


def _deg_partials(dst3, ew3, n_pad, c_per_tile):
    @functools.partial(
        pl.kernel,
        out_type=jax.ShapeDtypeStruct((NC, n_pad), jnp.float32),
        mesh=_sc_mesh(),
        scratch_types=[
            pltpu.VMEM((c_per_tile, K), jnp.int32),
            pltpu.VMEM((c_per_tile, K), jnp.float32),
            pltpu.VMEM_SHARED((n_pad,), jnp.float32),
        ],
    )
    def deg_kernel(dst_hbm, ew_hbm, out_hbm, dst_v, ew_v, acc):
        cid = lax.axis_index("c")
        sid = lax.axis_index("s")
        wid = cid * NS + sid
        sl_per_tile = n_pad // NS
        base = sid * sl_per_tile

        zeros16 = jnp.zeros((LANES,), jnp.float32)

        @pl.loop(0, K // LANES)
        def _(i):
            ew_v[0, pl.ds(i * LANES, LANES)] = zeros16

        @pl.loop(0, sl_per_tile // K)
        def _(t):
            pltpu.sync_copy(ew_v.at[0], acc.at[pl.ds(base + t * K, K)])

        rem = sl_per_tile % K
        if rem:
            pltpu.sync_copy(ew_v.at[0, pl.ds(0, rem)],
                            acc.at[pl.ds(base + (sl_per_tile // K) * K, rem)])

        plsc.subcore_barrier()

        pltpu.sync_copy(dst_hbm.at[wid], dst_v)
        pltpu.sync_copy(ew_hbm.at[wid], ew_v)

        @pl.loop(0, c_per_tile)
        def _(c):
            pltpu.sync_copy(ew_v.at[c], acc.at[dst_v.at[c]], add=True)

        plsc.subcore_barrier()
        pltpu.sync_copy(acc.at[pl.ds(base, sl_per_tile)],
                        out_hbm.at[cid, pl.ds(base, sl_per_tile)])

    return deg_kernel(dst3, ew3)


RB = 4
EB = 6
UNROLL = 12


def _aggregate(hp, meta, n, d, c_per_tile):
    assert c_per_tile % UNROLL == 0
    rows_per_tile = n // NS

    @functools.partial(
        pl.kernel,
        out_type=jax.ShapeDtypeStruct((NC, NS, rows_per_tile, d), jnp.float32),
        mesh=_sc_mesh(),
        scratch_types=[
            pltpu.VMEM((EB, 3, K), jnp.int32),
            [pltpu.VMEM((K, d), jnp.float32)] * RB,
            pltpu.VMEM_SHARED((n, d), jnp.float32),
            [pltpu.SemaphoreType.DMA] * EB,
            [pltpu.SemaphoreType.DMA] * RB,
            [pltpu.SemaphoreType.DMA] * RB,
        ],
    )
    def agg_kernel(hp_hbm, meta_hbm, out_hbm,
                   meta_v, rows, acc, esem, gsem, ssem):
        rows_v = rows[0]
        cid = lax.axis_index("c")
        sid = lax.axis_index("s")
        wid = cid * NS + sid
        base = sid * rows_per_tile
        zeros16 = jnp.zeros((LANES,), jnp.float32)

        @pl.loop(0, K)
        def _(e):
            for j in range(d // LANES):
                rows_v[e, pl.ds(j * LANES, LANES)] = zeros16

        @pl.loop(0, rows_per_tile // K)
        def _(t):
            pltpu.sync_copy(rows_v, acc.at[pl.ds(base + t * K, K)])

        rem = rows_per_tile % K
        if rem:
            pltpu.sync_copy(rows_v.at[pl.ds(0, rem)],
                            acc.at[pl.ds(base + (rows_per_tile // K) * K, rem)])

        plsc.subcore_barrier()

        def edges_desc(c, sl):
            return pltpu.make_async_copy(meta_hbm.at[wid, c], meta_v.at[sl],
                                         esem[sl])

        def gather_desc(b, sl):
            return pltpu.make_async_copy(hp_hbm.at[meta_v.at[sl, 0]], rows[b],
                                         gsem[b])

        def scatter_desc(b, sl):
            return pltpu.make_async_copy(rows[b], acc.at[meta_v.at[sl, 1]],
                                         ssem[b])

        edges_desc(0, 0).start()
        edges_desc(1, 1).start()
        edges_desc(2, 2).start()
        edges_desc(0, 0).wait()
        gather_desc(0, 0).start()
        edges_desc(1, 1).wait()
        gather_desc(1, 1).start()

        @pl.loop(0, c_per_tile // UNROLL)
        def _(t):
            for r in range(UNROLL):
                b = r % RB
                b2 = (r + 2) % RB
                sl = r % EB
                sl2 = (r + 2) % EB
                sl3 = (r + 3) % EB
                c = t * UNROLL + r

                @pl.when(c >= 2)
                def _():
                    scatter_desc(b2, sl2).wait()

                @pl.when(c + 2 < c_per_tile)
                def _():
                    edges_desc(c + 2, sl2).wait()
                    gather_desc(b2, sl2).start()

                @pl.when(c + 3 < c_per_tile)
                def _():
                    edges_desc(c + 3, sl3).start()

                gather_desc(b, sl).wait()

                @pl.loop(0, K // LANES)
                def _(i):
                    ew_vec = lax.bitcast_convert_type(
                        meta_v[sl, 2, pl.ds(i * LANES, LANES)], jnp.float32)
                    for l in range(LANES):
                        s = _splat_lane(ew_vec, l)
                        e = i * LANES + l
                        for j in range(d // LANES):
                            csl = pl.ds(j * LANES, LANES)
                            rows[b][e, csl] = rows[b][e, csl] * s

                scatter_desc(b, sl).start(add=True)

        for c_last in (c_per_tile - 2, c_per_tile - 1):
            scatter_desc(c_last % RB, c_last % EB).wait()

        plsc.subcore_barrier()
        pltpu.sync_copy(acc.at[pl.ds(base, rows_per_tile)],
                        out_hbm.at[cid, sid])

    return agg_kernel(hp, meta)


_BLK = 10000


def _dinv_block(degp_ref):
    deg = degp_ref[:, 0] + degp_ref[:, 1] + 1.0
    return lax.rsqrt(deg)[:, None]


def _prep_body(x_ref, w_ref, degp_ref, hp_ref):
    h = jnp.dot(x_ref[...], w_ref[...], preferred_element_type=jnp.float32)
    hp_ref[...] = h * _dinv_block(degp_ref)


def _mid_body(sp_ref, hp_ref, degp_ref, b_ref, w_ref, hp2_ref):
    dinv = _dinv_block(degp_ref)
    s = sp_ref[0] + sp_ref[1] + hp_ref[...]
    out1 = jnp.maximum(dinv * s + b_ref[...], 0.0)
    h2 = jnp.dot(out1, w_ref[...], preferred_element_type=jnp.float32)
    hp2_ref[...] = h2 * dinv


def _final_body(sp_ref, hp_ref, degp_ref, b_ref, out_ref):
    dinv = _dinv_block(degp_ref)
    s = sp_ref[0] + sp_ref[1] + hp_ref[...]
    out_ref[...] = jnp.maximum(dinv * s + b_ref[...], 0.0)


def _row_grid(n, d):
    grid = n // _BLK
    nd_spec = pl.BlockSpec((_BLK, d), lambda i: (i, 0))
    p_spec = pl.BlockSpec((2, _BLK, d), lambda i: (0, i, 0))
    deg_spec = pl.BlockSpec((_BLK, 2), lambda i: (i, 0))
    w_spec = pl.BlockSpec((d, d), lambda i: (0, 0))
    b_spec = pl.BlockSpec((1, d), lambda i: (0, 0))
    return grid, nd_spec, p_spec, deg_spec, w_spec, b_spec


def kernel(x, edge_idx, edge_attr, W1, b1, W2, b2):
    n, d = x.shape
    e = edge_attr.shape[0]

    chunk_all = NW * K * UNROLL
    e_pad = ((e + chunk_all - 1) // chunk_all) * chunk_all
    c_per_tile = e_pad // (NW * K)
    npg = NS * 128
    n_pad = ((n + npg - 1) // npg) * npg

    pad = e_pad - e
    pad_idx = jnp.arange(pad, dtype=jnp.int32) % n
    src3 = jnp.concatenate([edge_idx[0], pad_idx]).reshape(NW, c_per_tile, K)
    dst3 = jnp.concatenate([edge_idx[1], pad_idx]).reshape(NW, c_per_tile, K)
    ew3 = jnp.concatenate(
        [edge_attr, jnp.zeros((pad,), jnp.float32)]).reshape(NW, c_per_tile, K)
    meta = jnp.stack(
        [src3, dst3, lax.bitcast_convert_type(ew3, jnp.int32)], axis=2)

    degp = _deg_partials(dst3, ew3, n_pad, c_per_tile)[:, :n].T

    grid, nd_spec, p_spec, deg_spec, w_spec, b_spec = _row_grid(n, d)
    out_nd = jax.ShapeDtypeStruct((n, d), jnp.float32)

    hp1 = pl.pallas_call(
        _prep_body, grid=grid,
        in_specs=[nd_spec, w_spec, deg_spec],
        out_specs=nd_spec, out_shape=out_nd,
    )(x, W1, degp)

    s1 = _aggregate(hp1, meta, n, d, c_per_tile).reshape(NC, n, d)

    hp2 = pl.pallas_call(
        _mid_body, grid=grid,
        in_specs=[p_spec, nd_spec, deg_spec, b_spec, w_spec],
        out_specs=nd_spec, out_shape=out_nd,
    )(s1, hp1, degp, b1.reshape(1, d), W2)

    s2 = _aggregate(hp2, meta, n, d, c_per_tile).reshape(NC, n, d)

    out = pl.pallas_call(
        _final_body, grid=grid,
        in_specs=[p_spec, nd_spec, deg_spec, b_spec],
        out_specs=nd_spec, out_shape=out_nd,
    )(s2, hp2, degp, b2.reshape(1, d))

    return out
